# baseline (speedup 1.0000x reference)
import functools

import jax
import jax.numpy as jnp
from jax import lax
from jax.experimental import pallas as pl
from jax.experimental.pallas import tpu as pltpu
from jax.experimental.pallas import tpu_sc as plsc

VOCAB = 100000
HALF = VOCAB // 2
LANES = 16


def _build(batch, seq):
    groups = (seq + LANES - 1) // LANES
    seq_pad = groups * LANES
    mesh = plsc.VectorSubcoreMesh(core_axis_name="c", subcore_axis_name="s")
    info = plsc.get_sparse_core_info()
    num_workers = info.num_cores * info.num_subcores
    rows_per_w = batch // num_workers

    @functools.partial(
        pl.kernel,
        mesh=mesh,
        out_type=jax.ShapeDtypeStruct((batch * VOCAB,), jnp.float32),
        scratch_types=[
            pltpu.VMEM((seq_pad,), jnp.int32),
            pltpu.VMEM((seq_pad,), jnp.float32),
            pltpu.VMEM((seq_pad,), jnp.int32),
            pltpu.VMEM((seq_pad,), jnp.float32),
            pltpu.VMEM((HALF,), jnp.float32),
            pltpu.VMEM((HALF,), jnp.float32),
            pltpu.VMEM((LANES,), jnp.int32),
            pltpu.VMEM((LANES,), jnp.float32),
            pltpu.SemaphoreType.DMA,
            pltpu.SemaphoreType.DMA,
            pltpu.SemaphoreType.DMA,
            pltpu.SemaphoreType.DMA,
        ],
        compiler_params=pltpu.CompilerParams(needs_layout_passes=False),
    )
    def pg_kernel(ids_hbm, w_hbm, out_hbm, idx0, wv0, idx1, wv1, buf_a, buf_b,
                  ibuf, fbuf, sem_a, sem_b, sem_i, sem_w):
        wid = lax.axis_index("s") * info.num_cores + lax.axis_index("c")
        base = wid * rows_per_w
        izeros = jnp.zeros((LANES,), jnp.int32)
        fzeros = jnp.zeros((LANES,), jnp.float32)
        iota = lax.iota(jnp.int32, LANES)
        prev_idx = jnp.maximum(iota - 1, 0)
        next_idx = jnp.minimum(iota + 1, LANES - 1)

        for g in range(groups):
            sl = pl.ds(g * LANES, LANES)
            idx0[sl] = izeros
            wv0[sl] = fzeros
            idx1[sl] = izeros
            wv1[sl] = fzeros

        def zero_body(i, carry):
            for j in range(8):
                sl = pl.ds((i * 8 + j) * LANES, LANES)
                buf_a[sl] = fzeros
                buf_b[sl] = fzeros
            return carry

        full = HALF // (LANES * 8)
        lax.fori_loop(0, full, zero_body, 0)
        for t in range(full * 8, HALF // LANES):
            sl = pl.ds(t * LANES, LANES)
            buf_a[sl] = fzeros
            buf_b[sl] = fzeros

        def stage(r, iv, wv):
            src_i = ids_hbm.at[pl.ds(r * seq, seq)]
            src_w = w_hbm.at[pl.ds(r * seq, seq)]
            pltpu.async_copy(src_i, iv.at[pl.ds(0, seq)], sem_i)
            pltpu.async_copy(src_w, wv.at[pl.ds(0, seq)], sem_w)
            pltpu.make_async_copy(src_i, iv.at[pl.ds(0, seq)], sem_i).wait()
            pltpu.make_async_copy(src_w, wv.at[pl.ds(0, seq)], sem_w).wait()

        def dedup(kk, vv):
            ks, vs = plsc.sort_key_val(kk, vv)
            ibuf[...] = ks
            prev = plsc.load_gather(ibuf, [prev_idx])
            knext = plsc.load_gather(ibuf, [next_idx])
            is_start = (iota == 0) | (ks != prev)
            is_end = (iota == LANES - 1) | (ks != knext)
            csum = plsc.cumsum(vs)
            fbuf[...] = csum
            startidx = plsc.cummax(jnp.where(is_start, iota, 0))
            cprev = plsc.load_gather(fbuf, [jnp.maximum(startidx - 1, 0)])
            seg = jnp.where(startidx == 0, csum, csum - cprev)
            return ks, seg, is_end

        def scatter_half(buf, iv, wv, lo):
            for g in range(groups):
                sl = pl.ds(g * LANES, LANES)
                ks, seg, is_end = dedup(iv[sl], wv[sl])
                inr = (ks >= lo) & (ks < lo + HALF)
                loc = jnp.clip(ks - lo, 0, HALF - 1)
                plsc.addupdate_scatter(buf, [loc], seg, mask=is_end & inr)

        def reset_half(buf, iv, lo):
            for g in range(groups):
                kk = iv[pl.ds(g * LANES, LANES)]
                inr = (kk >= lo) & (kk < lo + HALF)
                loc = jnp.clip(kk - lo, 0, HALF - 1)
                plsc.store_scatter(buf, [loc], fzeros, mask=inr)

        def start_dma(buf, r, lo, sem):
            pltpu.async_copy(buf, out_hbm.at[pl.ds(r * VOCAB + lo, HALF)],
                             sem)

        def wait_dma(buf, r, lo, sem):
            pltpu.make_async_copy(buf, out_hbm.at[pl.ds(r * VOCAB + lo, HALF)],
                                  sem).wait()

        stage(base, idx0, wv0)
        scatter_half(buf_a, idx0, wv0, 0)
        start_dma(buf_a, base, 0, sem_a)
        scatter_half(buf_b, idx0, wv0, HALF)
        start_dma(buf_b, base, HALF, sem_b)

        stage(base + 1, idx1, wv1)
        wait_dma(buf_a, base, 0, sem_a)
        reset_half(buf_a, idx0, 0)
        scatter_half(buf_a, idx1, wv1, 0)
        start_dma(buf_a, base + 1, 0, sem_a)
        wait_dma(buf_b, base, HALF, sem_b)
        reset_half(buf_b, idx0, HALF)
        scatter_half(buf_b, idx1, wv1, HALF)
        start_dma(buf_b, base + 1, HALF, sem_b)

        def pair_body(i, carry):
            ra = base + 2 * i
            rb = ra + 1
            stage(ra, idx0, wv0)
            wait_dma(buf_a, ra - 1, 0, sem_a)
            reset_half(buf_a, idx1, 0)
            scatter_half(buf_a, idx0, wv0, 0)
            start_dma(buf_a, ra, 0, sem_a)
            wait_dma(buf_b, ra - 1, HALF, sem_b)
            reset_half(buf_b, idx1, HALF)
            scatter_half(buf_b, idx0, wv0, HALF)
            start_dma(buf_b, ra, HALF, sem_b)

            stage(rb, idx1, wv1)
            wait_dma(buf_a, ra, 0, sem_a)
            reset_half(buf_a, idx0, 0)
            scatter_half(buf_a, idx1, wv1, 0)
            start_dma(buf_a, rb, 0, sem_a)
            wait_dma(buf_b, ra, HALF, sem_b)
            reset_half(buf_b, idx0, HALF)
            scatter_half(buf_b, idx1, wv1, HALF)
            start_dma(buf_b, rb, HALF, sem_b)
            return carry

        lax.fori_loop(1, rows_per_w // 2, pair_body, 0)

        last = base + rows_per_w - 1
        wait_dma(buf_a, last, 0, sem_a)
        wait_dma(buf_b, last, HALF, sem_b)

    return pg_kernel


def kernel(encoder_inputs, attention_weight):
    batch, seq = encoder_inputs.shape
    ids = encoder_inputs.astype(jnp.int32).reshape(-1)
    w = attention_weight.astype(jnp.float32).reshape(-1)
    return _build(batch, seq)(ids, w).reshape(batch, VOCAB)


# speedup vs baseline: 1.8129x; 1.8129x over previous
import functools

import jax
import jax.numpy as jnp
from jax import lax
from jax.experimental import pallas as pl
from jax.experimental.pallas import tpu as pltpu
from jax.experimental.pallas import tpu_sc as plsc

VOCAB = 100000
LANES = 16


def _build(batch, seq):
    groups = (seq + LANES - 1) // LANES
    seq_pad = groups * LANES
    mesh = plsc.VectorSubcoreMesh(core_axis_name="c", subcore_axis_name="s")
    info = plsc.get_sparse_core_info()
    num_workers = info.num_cores * info.num_subcores
    rows_per_w = batch // num_workers

    @functools.partial(
        pl.kernel,
        mesh=mesh,
        out_type=jax.ShapeDtypeStruct((batch, VOCAB), jnp.float32),
        scratch_types=[
            pltpu.VMEM((seq_pad,), jnp.int32),
            pltpu.VMEM((seq_pad,), jnp.float32),
            pltpu.VMEM((VOCAB,), jnp.float32),
        ],
        compiler_params=pltpu.CompilerParams(needs_layout_passes=False),
    )
    def pg_kernel(ids_hbm, w_hbm, out_hbm, idx_v, w_v, rowbuf):
        wid = lax.axis_index("s") * info.num_cores + lax.axis_index("c")
        izeros = jnp.zeros((LANES,), jnp.int32)
        fzeros = jnp.zeros((LANES,), jnp.float32)

        for g in range(groups):
            idx_v[pl.ds(g * LANES, LANES)] = izeros
            w_v[pl.ds(g * LANES, LANES)] = fzeros

        def zero_body(i, carry):
            rowbuf[pl.ds(i * LANES, LANES)] = fzeros
            return carry

        lax.fori_loop(0, VOCAB // LANES, zero_body, 0)

        def row_body(r0, carry):
            r = wid * rows_per_w + r0
            pltpu.sync_copy(ids_hbm.at[pl.ds(r * seq, seq)],
                            idx_v.at[pl.ds(0, seq)])
            pltpu.sync_copy(w_hbm.at[pl.ds(r * seq, seq)],
                            w_v.at[pl.ds(0, seq)])
            for g in range(groups):
                kk = idx_v[pl.ds(g * LANES, LANES)]
                vv = w_v[pl.ds(g * LANES, LANES)]
                plsc.addupdate_scatter(rowbuf, [kk], vv)
            pltpu.sync_copy(rowbuf, out_hbm.at[r])
            for g in range(groups):
                kk = idx_v[pl.ds(g * LANES, LANES)]
                plsc.store_scatter(rowbuf, [kk], fzeros)
            return carry

        lax.fori_loop(0, rows_per_w, row_body, 0)

    return pg_kernel


def kernel(encoder_inputs, attention_weight):
    batch, seq = encoder_inputs.shape
    ids = encoder_inputs.astype(jnp.int32).reshape(-1)
    w = attention_weight.astype(jnp.float32).reshape(-1)
    return _build(batch, seq)(ids, w)


# speedup vs baseline: 1.9048x; 1.0507x over previous
import functools

import jax
import jax.numpy as jnp
from jax import lax
from jax.experimental import pallas as pl
from jax.experimental.pallas import tpu as pltpu
from jax.experimental.pallas import tpu_sc as plsc

VOCAB = 100000
LANES = 16


def _build(batch, seq):
    groups = (seq + LANES - 1) // LANES
    seq_pad = groups * LANES
    mesh = plsc.VectorSubcoreMesh(core_axis_name="c", subcore_axis_name="s")
    info = plsc.get_sparse_core_info()
    num_workers = info.num_cores * info.num_subcores
    rows_per_w = batch // num_workers

    @functools.partial(
        pl.kernel,
        mesh=mesh,
        out_type=jax.ShapeDtypeStruct((batch, VOCAB), jnp.float32),
        scratch_types=[
            pltpu.VMEM((seq_pad,), jnp.int32),
            pltpu.VMEM((seq_pad,), jnp.float32),
            pltpu.VMEM((seq_pad,), jnp.int32),
            pltpu.VMEM((seq_pad,), jnp.float32),
            pltpu.VMEM((VOCAB,), jnp.float32),
            pltpu.SemaphoreType.DMA,
            pltpu.SemaphoreType.DMA,
            pltpu.SemaphoreType.DMA,
            pltpu.SemaphoreType.DMA,
            pltpu.SemaphoreType.DMA,
        ],
        compiler_params=pltpu.CompilerParams(needs_layout_passes=False),
    )
    def pg_kernel(ids_hbm, w_hbm, out_hbm, idx0, wv0, idx1, wv1, rowbuf,
                  sem_out, sem_i0, sem_w0, sem_i1, sem_w1):
        wid = lax.axis_index("s") * info.num_cores + lax.axis_index("c")
        base = wid * rows_per_w
        izeros = jnp.zeros((LANES,), jnp.int32)
        fzeros = jnp.zeros((LANES,), jnp.float32)

        for g in range(groups):
            sl = pl.ds(g * LANES, LANES)
            idx0[sl] = izeros
            wv0[sl] = fzeros
            idx1[sl] = izeros
            wv1[sl] = fzeros

        def zero_body(i, carry):
            rowbuf[pl.ds(i * LANES, LANES)] = fzeros
            return carry

        lax.fori_loop(0, VOCAB // LANES, zero_body, 0)

        def stage_start(r, iv, wv, si, sw):
            pltpu.async_copy(ids_hbm.at[pl.ds(r * seq, seq)],
                             iv.at[pl.ds(0, seq)], si)
            pltpu.async_copy(w_hbm.at[pl.ds(r * seq, seq)],
                             wv.at[pl.ds(0, seq)], sw)

        def stage_wait(r, iv, wv, si, sw):
            pltpu.make_async_copy(ids_hbm.at[pl.ds(r * seq, seq)],
                                  iv.at[pl.ds(0, seq)], si).wait()
            pltpu.make_async_copy(w_hbm.at[pl.ds(r * seq, seq)],
                                  wv.at[pl.ds(0, seq)], sw).wait()

        def scatter(iv, wv):
            for g in range(groups):
                sl = pl.ds(g * LANES, LANES)
                plsc.addupdate_scatter(rowbuf, [iv[sl]], wv[sl])

        def reset(iv):
            for g in range(groups):
                plsc.store_scatter(rowbuf, [iv[pl.ds(g * LANES, LANES)]],
                                   fzeros)

        def out_start(r):
            pltpu.async_copy(rowbuf, out_hbm.at[r], sem_out)

        def out_wait(r):
            pltpu.make_async_copy(rowbuf, out_hbm.at[r], sem_out).wait()

        stage_start(base, idx0, wv0, sem_i0, sem_w0)
        stage_wait(base, idx0, wv0, sem_i0, sem_w0)
        scatter(idx0, wv0)
        out_start(base)
        stage_start(base + 1, idx1, wv1, sem_i1, sem_w1)

        def pair_body(i, carry):
            ra = base + 2 * i + 1
            rb = ra + 1
            out_wait(ra - 1)
            reset(idx0)
            stage_start(rb, idx0, wv0, sem_i0, sem_w0)
            stage_wait(ra, idx1, wv1, sem_i1, sem_w1)
            scatter(idx1, wv1)
            out_start(ra)
            out_wait(ra)
            reset(idx1)

            @pl.when(rb + 1 < base + rows_per_w)
            def _():
                stage_start(rb + 1, idx1, wv1, sem_i1, sem_w1)

            stage_wait(rb, idx0, wv0, sem_i0, sem_w0)
            scatter(idx0, wv0)
            out_start(rb)
            return carry

        lax.fori_loop(0, (rows_per_w - 1) // 2, pair_body, 0)

        last = base + rows_per_w - 1
        out_wait(last - 1)
        reset(idx0)
        stage_wait(last, idx1, wv1, sem_i1, sem_w1)
        scatter(idx1, wv1)
        out_start(last)
        out_wait(last)

    return pg_kernel


def kernel(encoder_inputs, attention_weight):
    batch, seq = encoder_inputs.shape
    ids = encoder_inputs.astype(jnp.int32).reshape(-1)
    w = attention_weight.astype(jnp.float32).reshape(-1)
    return _build(batch, seq)(ids, w)
